# baseline (speedup 1.0000x reference)
import jax
import jax.numpy as jnp
from jax.experimental import pallas as pl


def _logits_kernel(feat_ref, b_feat_ref, w_feat_ref, w_lin_ref, b_lin_ref,
                   out_ref):
    w_small = jnp.dot(w_feat_ref[...], w_lin_ref[...],
                      preferred_element_type=jnp.float32)
    x = feat_ref[...] + b_feat_ref[...]
    out_ref[...] = (jnp.dot(x, w_small, preferred_element_type=jnp.float32)
                    + b_lin_ref[...])


def kernel(feat, edge_index, b_feat, W_feat, W_n, b_n, W_lin, b_lin):
    del edge_index, W_n, b_n
    n, f = feat.shape
    h = W_feat.shape[1]
    grid = 10
    blk = n // grid
    b_lin2 = b_lin.reshape(1, 1)
    return pl.pallas_call(
        _logits_kernel,
        grid=(grid,),
        in_specs=[
            pl.BlockSpec((blk, f), lambda i: (i, 0)),
            pl.BlockSpec(b_feat.shape, lambda i: (0, 0)),
            pl.BlockSpec((f, h), lambda i: (0, 0)),
            pl.BlockSpec((h, 1), lambda i: (0, 0)),
            pl.BlockSpec((1, 1), lambda i: (0, 0)),
        ],
        out_specs=pl.BlockSpec((blk, 1), lambda i: (i, 0)),
        out_shape=jax.ShapeDtypeStruct((n, 1), jnp.float32),
    )(feat, b_feat, W_feat, W_lin, b_lin2)
